# initial kernel scaffold (unmeasured)
import jax
import jax.numpy as jnp
from jax import lax
from jax.experimental import pallas as pl
from jax.experimental.pallas import tpu as pltpu

S = 1024
D = 2048
DC = 256
DC_SH = 128
H = 16
DH = 128
DR = 32
SCALE = (DH + DR) ** -0.5


def _kv_body(x_ref, wdkv_ref, wuk_ref, wuv_ref, wkr_ref,
             k_ref, v_ref, kr_ref,
             wdkv_o, wuk_o, wuv_o, send_sems, recv_sems):
    my_x = lax.axis_index("x")
    my_y = lax.axis_index("y")
    other = (1 - my_x, my_y)

    copies = []
    for i, (src, dst) in enumerate(
        ((wdkv_ref, wdkv_o), (wuk_ref, wuk_o), (wuv_ref, wuv_o))
    ):
        c = pltpu.make_async_remote_copy(
            src_ref=src,
            dst_ref=dst,
            send_sem=send_sems.at[i],
            recv_sem=recv_sems.at[i],
            device_id=other,
            device_id_type=pl.DeviceIdType.MESH,
        )
        c.start()
        copies.append(c)

    x = x_ref[...]
    kr_ref[...] = jnp.dot(x, wkr_ref[...], preferred_element_type=jnp.float32)
    c_mine = jnp.dot(x, wdkv_ref[...], preferred_element_type=jnp.float32)

    for c in copies:
        c.wait()

    c_oth = jnp.dot(x, wdkv_o[...], preferred_element_type=jnp.float32)
    k_ref[...] = (
        jnp.dot(c_mine, wuk_ref[...], preferred_element_type=jnp.float32)
        + jnp.dot(c_oth, wuk_o[...], preferred_element_type=jnp.float32)
    )
    v_ref[...] = (
        jnp.dot(c_mine, wuv_ref[...], preferred_element_type=jnp.float32)
        + jnp.dot(c_oth, wuv_o[...], preferred_element_type=jnp.float32)
    )


def _attn_body(x_ref, wq_ref, wqr_ref, k_ref, v_ref, kr_ref, o_ref):
    q = jnp.dot(x_ref[...], wq_ref[...], preferred_element_type=jnp.float32)
    qr = jnp.dot(x_ref[...], wqr_ref[...], preferred_element_type=jnp.float32)
    s = lax.dot_general(
        q, k_ref[...], (((1,), (1,)), ((), ())),
        preferred_element_type=jnp.float32,
    )
    s = s + lax.dot_general(
        qr, kr_ref[...], (((1,), (1,)), ((), ())),
        preferred_element_type=jnp.float32,
    )
    s = s * SCALE
    m = jnp.max(s, axis=1, keepdims=True)
    p = jnp.exp(s - m)
    p = p / jnp.sum(p, axis=1, keepdims=True)
    o_ref[...] = jnp.dot(p, v_ref[...], preferred_element_type=jnp.float32)


def _out_body(o_ref, wo_ref, y_ref):
    y_ref[...] = jnp.dot(
        o_ref[...], wo_ref[...], preferred_element_type=jnp.float32
    )


def kernel(x, Wdkv, Wuk, Wuv, Wq, Wqr, Wkr, Wo):
    x2 = x.reshape(S, D)

    k, v, kr = pl.pallas_call(
        _kv_body,
        out_shape=(
            jax.ShapeDtypeStruct((S, D), jnp.float32),
            jax.ShapeDtypeStruct((S, D), jnp.float32),
            jax.ShapeDtypeStruct((S, DR), jnp.float32),
        ),
        in_specs=[pl.BlockSpec(memory_space=pltpu.VMEM)] * 5,
        out_specs=(pl.BlockSpec(memory_space=pltpu.VMEM),) * 3,
        scratch_shapes=[
            pltpu.VMEM((D, DC_SH), jnp.float32),
            pltpu.VMEM((DC_SH, D), jnp.float32),
            pltpu.VMEM((DC_SH, D), jnp.float32),
            pltpu.SemaphoreType.DMA((3,)),
            pltpu.SemaphoreType.DMA((3,)),
        ],
    )(x2, Wdkv, Wuk, Wuv, Wkr)

    o = pl.pallas_call(
        _attn_body,
        grid=(H,),
        out_shape=jax.ShapeDtypeStruct((S, D), jnp.float32),
        in_specs=[
            pl.BlockSpec((S, D), lambda h: (0, 0)),
            pl.BlockSpec((D, DH), lambda h: (0, h)),
            pl.BlockSpec((D, DR), lambda h: (0, h)),
            pl.BlockSpec((S, DH), lambda h: (0, h)),
            pl.BlockSpec((S, DH), lambda h: (0, h)),
            pl.BlockSpec((S, DR), lambda h: (0, 0)),
        ],
        out_specs=pl.BlockSpec((S, DH), lambda h: (0, h)),
    )(x2, Wq, Wqr, k, v, kr)

    y = pl.pallas_call(
        _out_body,
        out_shape=jax.ShapeDtypeStruct((S, D), jnp.float32),
        in_specs=[pl.BlockSpec(memory_space=pltpu.VMEM)] * 2,
        out_specs=pl.BlockSpec(memory_space=pltpu.VMEM),
    )(o, Wo)

    return y.reshape(1, S, D)


# baseline (device time: 152642 ns/iter reference)
import jax
import jax.numpy as jnp
from jax import lax
from jax.experimental import pallas as pl
from jax.experimental.pallas import tpu as pltpu

S = 1024
D = 2048
DC = 256
DC_SH = 128
H = 16
DH = 128
DR = 32
SCALE = (DH + DR) ** -0.5


def _kv_body(x_ref, wdkv_ref, wuk_ref, wuv_ref, wkr_ref,
             k_ref, v_ref, kr_ref,
             wdkv_o, wuk_o, wuv_o, send_sems, recv_sems):
    my_x = lax.axis_index("x")
    my_y = lax.axis_index("y")
    other = (1 - my_x, my_y)

    copies = []
    for i, (src, dst) in enumerate(
        ((wdkv_ref, wdkv_o), (wuk_ref, wuk_o), (wuv_ref, wuv_o))
    ):
        c = pltpu.make_async_remote_copy(
            src_ref=src,
            dst_ref=dst,
            send_sem=send_sems.at[i],
            recv_sem=recv_sems.at[i],
            device_id=other,
            device_id_type=pl.DeviceIdType.MESH,
        )
        c.start()
        copies.append(c)

    x = x_ref[...]
    kr_ref[...] = jnp.dot(x, wkr_ref[...], preferred_element_type=jnp.float32)
    c_mine = jnp.dot(x, wdkv_ref[...], preferred_element_type=jnp.float32)

    for c in copies:
        c.wait()

    c_oth = jnp.dot(x, wdkv_o[...], preferred_element_type=jnp.float32)
    k_ref[...] = (
        jnp.dot(c_mine, wuk_ref[...], preferred_element_type=jnp.float32)
        + jnp.dot(c_oth, wuk_o[...], preferred_element_type=jnp.float32)
    )
    v_ref[...] = (
        jnp.dot(c_mine, wuv_ref[...], preferred_element_type=jnp.float32)
        + jnp.dot(c_oth, wuv_o[...], preferred_element_type=jnp.float32)
    )


HG = 4


def _attn_body(x_ref, wq_ref, wqr_ref, k_ref, v_ref, kr_ref, o_ref):
    x = x_ref[...]
    q = jnp.dot(x, wq_ref[...], preferred_element_type=jnp.float32)
    qr = jnp.dot(x, wqr_ref[...], preferred_element_type=jnp.float32)
    kr = kr_ref[...]
    for i in range(HG):
        qi = q[:, i * DH:(i + 1) * DH]
        ki = k_ref[:, i * DH:(i + 1) * DH]
        qri = qr[:, i * DR:(i + 1) * DR]
        s = lax.dot_general(
            qi, ki, (((1,), (1,)), ((), ())),
            preferred_element_type=jnp.float32,
        )
        s = s + lax.dot_general(
            qri, kr, (((1,), (1,)), ((), ())),
            preferred_element_type=jnp.float32,
        )
        s = s * SCALE
        m = jnp.max(s, axis=1, keepdims=True)
        p = jnp.exp(s - m)
        p = p / jnp.sum(p, axis=1, keepdims=True)
        vi = v_ref[:, i * DH:(i + 1) * DH]
        o_ref[:, i * DH:(i + 1) * DH] = jnp.dot(
            p, vi, preferred_element_type=jnp.float32
        )


def _out_body(o_ref, wo_ref, y_ref):
    y_ref[...] = jnp.dot(
        o_ref[...], wo_ref[...], preferred_element_type=jnp.float32
    )


def kernel(x, Wdkv, Wuk, Wuv, Wq, Wqr, Wkr, Wo):
    x2 = x.reshape(S, D)

    k, v, kr = pl.pallas_call(
        _kv_body,
        out_shape=(
            jax.ShapeDtypeStruct((S, D), jnp.float32),
            jax.ShapeDtypeStruct((S, D), jnp.float32),
            jax.ShapeDtypeStruct((S, DR), jnp.float32),
        ),
        in_specs=[pl.BlockSpec(memory_space=pltpu.VMEM)] * 5,
        out_specs=(pl.BlockSpec(memory_space=pltpu.VMEM),) * 3,
        scratch_shapes=[
            pltpu.VMEM((D, DC_SH), jnp.float32),
            pltpu.VMEM((DC_SH, D), jnp.float32),
            pltpu.VMEM((DC_SH, D), jnp.float32),
            pltpu.SemaphoreType.DMA((3,)),
            pltpu.SemaphoreType.DMA((3,)),
        ],
    )(x2, Wdkv, Wuk, Wuv, Wkr)

    o = pl.pallas_call(
        _attn_body,
        grid=(H // HG,),
        out_shape=jax.ShapeDtypeStruct((S, D), jnp.float32),
        in_specs=[
            pl.BlockSpec((S, D), lambda g: (0, 0)),
            pl.BlockSpec((D, HG * DH), lambda g: (0, g)),
            pl.BlockSpec((D, HG * DR), lambda g: (0, g)),
            pl.BlockSpec((S, HG * DH), lambda g: (0, g)),
            pl.BlockSpec((S, HG * DH), lambda g: (0, g)),
            pl.BlockSpec((S, DR), lambda g: (0, 0)),
        ],
        out_specs=pl.BlockSpec((S, HG * DH), lambda g: (0, g)),
    )(x2, Wq, Wqr, k, v, kr)

    y = pl.pallas_call(
        _out_body,
        out_shape=jax.ShapeDtypeStruct((S, D), jnp.float32),
        in_specs=[pl.BlockSpec(memory_space=pltpu.VMEM)] * 2,
        out_specs=pl.BlockSpec(memory_space=pltpu.VMEM),
    )(o, Wo)

    return y.reshape(1, S, D)


# device time: 150814 ns/iter; 1.0121x vs baseline; 1.0121x over previous
import jax
import jax.numpy as jnp
from jax import lax
from jax.experimental import pallas as pl
from jax.experimental.pallas import tpu as pltpu

S = 1024
D = 2048
DC = 256
DC_SH = 128
H = 16
DH = 128
DR = 32
SCALE = (DH + DR) ** -0.5


def _kv_body(x_ref, wdkv_ref, wuk_ref, wuv_ref, wkr_ref,
             k_ref, v_ref, kr_ref,
             wdkv_o, wuk_o, wuv_o, send_sems, recv_sems):
    my_x = lax.axis_index("x")
    my_y = lax.axis_index("y")
    other = (1 - my_x, my_y)

    copies = []
    for i, (src, dst) in enumerate(
        ((wdkv_ref, wdkv_o), (wuk_ref, wuk_o), (wuv_ref, wuv_o))
    ):
        c = pltpu.make_async_remote_copy(
            src_ref=src,
            dst_ref=dst,
            send_sem=send_sems.at[i],
            recv_sem=recv_sems.at[i],
            device_id=other,
            device_id_type=pl.DeviceIdType.MESH,
        )
        c.start()
        copies.append(c)

    x = x_ref[...]
    kr_ref[...] = jnp.dot(x, wkr_ref[...], preferred_element_type=jnp.float32)
    c_mine = jnp.dot(x, wdkv_ref[...], preferred_element_type=jnp.float32)

    for c in copies:
        c.wait()

    c_oth = jnp.dot(x, wdkv_o[...], preferred_element_type=jnp.float32)
    k_ref[...] = (
        jnp.dot(c_mine, wuk_ref[...], preferred_element_type=jnp.float32)
        + jnp.dot(c_oth, wuk_o[...], preferred_element_type=jnp.float32)
    )
    v_ref[...] = (
        jnp.dot(c_mine, wuv_ref[...], preferred_element_type=jnp.float32)
        + jnp.dot(c_oth, wuv_o[...], preferred_element_type=jnp.float32)
    )


HG = 4
S4 = S // 4


def _attn_body(x_ref, wq_ref, wqr_ref, k_ref, v_ref, kr_ref, o_ref):
    my_x = lax.axis_index("x")
    my_y = lax.axis_index("y")
    p = my_x * 2 + my_y
    x = x_ref[pl.ds(p * S4, S4), :]
    q = jnp.dot(x, wq_ref[...], preferred_element_type=jnp.float32)
    qr = jnp.dot(x, wqr_ref[...], preferred_element_type=jnp.float32)
    kr = kr_ref[...]
    for i in range(HG):
        qi = q[:, i * DH:(i + 1) * DH]
        ki = k_ref[:, i * DH:(i + 1) * DH]
        qri = qr[:, i * DR:(i + 1) * DR]
        s = lax.dot_general(
            qi, ki, (((1,), (1,)), ((), ())),
            preferred_element_type=jnp.float32,
        )
        s = s + lax.dot_general(
            qri, kr, (((1,), (1,)), ((), ())),
            preferred_element_type=jnp.float32,
        )
        s = s * SCALE
        m = jnp.max(s, axis=1, keepdims=True)
        p = jnp.exp(s - m)
        p = p / jnp.sum(p, axis=1, keepdims=True)
        vi = v_ref[:, i * DH:(i + 1) * DH]
        o_ref[:, i * DH:(i + 1) * DH] = jnp.dot(
            p, vi, preferred_element_type=jnp.float32
        )


def _out_body(o_ref, wo_ref, out_ref, send_sems, recv_sems):
    my_x = lax.axis_index("x")
    my_y = lax.axis_index("y")
    p = my_x * 2 + my_y
    mine = jnp.dot(o_ref[...], wo_ref[...], preferred_element_type=jnp.float32)
    out_ref[p] = mine

    peers = [(1 - my_x, my_y), (my_x, 1 - my_y), (1 - my_x, 1 - my_y)]
    sends = []
    for i, peer in enumerate(peers):
        r = pltpu.make_async_remote_copy(
            src_ref=out_ref.at[p],
            dst_ref=out_ref.at[p],
            send_sem=send_sems.at[i],
            recv_sem=recv_sems.at[p],
            device_id=peer,
            device_id_type=pl.DeviceIdType.MESH,
        )
        r.start()
        sends.append(r)
    for r in sends:
        r.wait_send()

    peer_slots = [(1 - my_x) * 2 + my_y, my_x * 2 + (1 - my_y),
                  (1 - my_x) * 2 + (1 - my_y)]
    for i, qp in enumerate(peer_slots):
        r = pltpu.make_async_remote_copy(
            src_ref=out_ref.at[qp],
            dst_ref=out_ref.at[qp],
            send_sem=send_sems.at[i],
            recv_sem=recv_sems.at[qp],
            device_id=(my_x, my_y),
            device_id_type=pl.DeviceIdType.MESH,
        )
        r.wait_recv()


def kernel(x, Wdkv, Wuk, Wuv, Wq, Wqr, Wkr, Wo):
    x2 = x.reshape(S, D)

    k, v, kr = pl.pallas_call(
        _kv_body,
        out_shape=(
            jax.ShapeDtypeStruct((S, D), jnp.float32),
            jax.ShapeDtypeStruct((S, D), jnp.float32),
            jax.ShapeDtypeStruct((S, DR), jnp.float32),
        ),
        in_specs=[pl.BlockSpec(memory_space=pltpu.VMEM)] * 5,
        out_specs=(pl.BlockSpec(memory_space=pltpu.VMEM),) * 3,
        scratch_shapes=[
            pltpu.VMEM((D, DC_SH), jnp.float32),
            pltpu.VMEM((DC_SH, D), jnp.float32),
            pltpu.VMEM((DC_SH, D), jnp.float32),
            pltpu.SemaphoreType.DMA((3,)),
            pltpu.SemaphoreType.DMA((3,)),
        ],
    )(x2, Wdkv, Wuk, Wuv, Wkr)

    o = pl.pallas_call(
        _attn_body,
        grid=(H // HG,),
        out_shape=jax.ShapeDtypeStruct((S4, D), jnp.float32),
        in_specs=[
            pl.BlockSpec((S, D), lambda g: (0, 0)),
            pl.BlockSpec((D, HG * DH), lambda g: (0, g)),
            pl.BlockSpec((D, HG * DR), lambda g: (0, g)),
            pl.BlockSpec((S, HG * DH), lambda g: (0, g)),
            pl.BlockSpec((S, HG * DH), lambda g: (0, g)),
            pl.BlockSpec((S, DR), lambda g: (0, 0)),
        ],
        out_specs=pl.BlockSpec((S4, HG * DH), lambda g: (0, g)),
    )(x2, Wq, Wqr, k, v, kr)

    y = pl.pallas_call(
        _out_body,
        out_shape=jax.ShapeDtypeStruct((4, S4, D), jnp.float32),
        in_specs=[pl.BlockSpec(memory_space=pltpu.VMEM)] * 2,
        out_specs=pl.BlockSpec(memory_space=pltpu.VMEM),
        scratch_shapes=[
            pltpu.SemaphoreType.DMA((3,)),
            pltpu.SemaphoreType.DMA((4,)),
        ],
    )(o, Wo)

    return y.reshape(1, S, D)


# device time: 138295 ns/iter; 1.1037x vs baseline; 1.0905x over previous
import jax
import jax.numpy as jnp
from jax import lax
from jax.experimental import pallas as pl
from jax.experimental.pallas import tpu as pltpu

S = 1024
D = 2048
DC_SH = 128
H = 16
DH = 128
DR = 32
SCALE = (DH + DR) ** -0.5
S4 = S // 4
HG = 4
NG = H // HG

F32 = jnp.float32


def _dot(a, b):
    return jnp.dot(a, b, preferred_element_type=F32)


def _dot_t(a, b):
    return lax.dot_general(a, b, (((1,), (1,)), ((), ())),
                           preferred_element_type=F32)


def _kv_body(x_ref, wdkv_ref, wuk_ref, wuv_ref, wkr_ref,
             k_ref, v_ref, kr_ref,
             c_mine_s, c_oth_s, wuk_o, wuv_o, w_send, w_recv):
    my_x = lax.axis_index("x")
    my_y = lax.axis_index("y")
    xnbr = (1 - my_x, my_y)

    barrier = pltpu.get_barrier_semaphore()
    pl.semaphore_signal(barrier, inc=1, device_id=xnbr,
                        device_id_type=pl.DeviceIdType.MESH)
    pl.semaphore_wait(barrier, 1)

    r_wuk = pltpu.make_async_remote_copy(
        src_ref=wuk_ref, dst_ref=wuk_o,
        send_sem=w_send.at[0], recv_sem=w_recv.at[0],
        device_id=xnbr, device_id_type=pl.DeviceIdType.MESH)
    r_wuk.start()

    x = x_ref[...]
    c_mine = _dot(x, wdkv_ref[...])
    c_mine_s[...] = c_mine
    r_c = pltpu.make_async_remote_copy(
        src_ref=c_mine_s, dst_ref=c_oth_s,
        send_sem=w_send.at[1], recv_sem=w_recv.at[1],
        device_id=xnbr, device_id_type=pl.DeviceIdType.MESH)
    r_c.start()
    r_wuv = pltpu.make_async_remote_copy(
        src_ref=wuv_ref, dst_ref=wuv_o,
        send_sem=w_send.at[2], recv_sem=w_recv.at[2],
        device_id=xnbr, device_id_type=pl.DeviceIdType.MESH)
    r_wuv.start()

    kr_ref[...] = _dot(x, wkr_ref[...])
    k_part = _dot(c_mine, wuk_ref[...])
    v_part = _dot(c_mine, wuv_ref[...])

    r_wuk.wait()
    r_c.wait()
    c_oth = c_oth_s[...]
    k_ref[...] = k_part + _dot(c_oth, wuk_o[...])
    r_wuv.wait()
    v_ref[...] = v_part + _dot(c_oth, wuv_o[...])


def _attn_body(x_ref, wq_ref, wqr_ref, k_ref, v_ref, kr_ref, wo_ref,
               out_ref, out_acc, copy_sem, ag_send, ag_recv):
    my_x = lax.axis_index("x")
    my_y = lax.axis_index("y")
    p = my_x * 2 + my_y
    peers = [(1 - my_x, my_y), (my_x, 1 - my_y), (1 - my_x, 1 - my_y)]
    peer_slots = [(1 - my_x) * 2 + my_y, my_x * 2 + (1 - my_y),
                  (1 - my_x) * 2 + (1 - my_y)]
    g = pl.program_id(0)
    barrier = pltpu.get_barrier_semaphore()

    @pl.when(g == 0)
    def _():
        for peer in peers:
            pl.semaphore_signal(barrier, inc=1, device_id=peer,
                                device_id_type=pl.DeviceIdType.MESH)

    x_mine = x_ref[pl.ds(p * S4, S4), :]
    q = _dot(x_mine, wq_ref[...])
    qr = _dot(x_mine, wqr_ref[...])
    kr = kr_ref[...]
    o_cols = []
    for i in range(HG):
        s = _dot_t(q[:, i * DH:(i + 1) * DH],
                   k_ref[:, i * DH:(i + 1) * DH])
        s = s + _dot_t(qr[:, i * DR:(i + 1) * DR], kr)
        s = s * SCALE
        m = jnp.max(s, axis=1, keepdims=True)
        pr = jnp.exp(s - m)
        pr = pr / jnp.sum(pr, axis=1, keepdims=True)
        o_cols.append(_dot(pr, v_ref[:, i * DH:(i + 1) * DH]))
    proj = _dot(jnp.concatenate(o_cols, axis=1), wo_ref[...])

    @pl.when(g == 0)
    def _():
        out_acc[...] = proj

    @pl.when(g != 0)
    def _():
        out_acc[...] = out_acc[...] + proj

    @pl.when(g == NG - 1)
    def _():
        pl.semaphore_wait(barrier, 3)
        cp = pltpu.make_async_copy(out_acc, out_ref.at[p], copy_sem)
        cp.start()
        cp.wait()
        sends = []
        for i, peer in enumerate(peers):
            r = pltpu.make_async_remote_copy(
                src_ref=out_ref.at[p], dst_ref=out_ref.at[p],
                send_sem=ag_send.at[i], recv_sem=ag_recv.at[p],
                device_id=peer, device_id_type=pl.DeviceIdType.MESH)
            r.start()
            sends.append(r)
        for r in sends:
            r.wait_send()
        for i, qp in enumerate(peer_slots):
            r = pltpu.make_async_remote_copy(
                src_ref=out_ref.at[qp], dst_ref=out_ref.at[qp],
                send_sem=ag_send.at[i], recv_sem=ag_recv.at[qp],
                device_id=(my_x, my_y), device_id_type=pl.DeviceIdType.MESH)
            r.wait_recv()


def kernel(x, Wdkv, Wuk, Wuv, Wq, Wqr, Wkr, Wo):
    x2 = x.reshape(S, D)

    k, v, kr = pl.pallas_call(
        _kv_body,
        out_shape=(
            jax.ShapeDtypeStruct((S, D), F32),
            jax.ShapeDtypeStruct((S, D), F32),
            jax.ShapeDtypeStruct((S, DR), F32),
        ),
        in_specs=[pl.BlockSpec(memory_space=pltpu.VMEM)] * 5,
        out_specs=(pl.BlockSpec(memory_space=pltpu.VMEM),) * 3,
        scratch_shapes=[
            pltpu.VMEM((S, DC_SH), F32),
            pltpu.VMEM((S, DC_SH), F32),
            pltpu.VMEM((DC_SH, D), F32),
            pltpu.VMEM((DC_SH, D), F32),
            pltpu.SemaphoreType.DMA((3,)),
            pltpu.SemaphoreType.DMA((3,)),
        ],
        compiler_params=pltpu.CompilerParams(collective_id=0),
    )(x2, Wdkv, Wuk, Wuv, Wkr)

    y = pl.pallas_call(
        _attn_body,
        grid=(NG,),
        out_shape=jax.ShapeDtypeStruct((4, S4, D), F32),
        in_specs=[
            pl.BlockSpec((S, D), lambda g: (0, 0)),
            pl.BlockSpec((D, HG * DH), lambda g: (0, g)),
            pl.BlockSpec((D, HG * DR), lambda g: (0, g)),
            pl.BlockSpec((S, HG * DH), lambda g: (0, g)),
            pl.BlockSpec((S, HG * DH), lambda g: (0, g)),
            pl.BlockSpec((S, DR), lambda g: (0, 0)),
            pl.BlockSpec((HG * DH, D), lambda g: (g, 0)),
        ],
        out_specs=pl.BlockSpec(memory_space=pl.ANY),
        scratch_shapes=[
            pltpu.VMEM((S4, D), F32),
            pltpu.SemaphoreType.DMA,
            pltpu.SemaphoreType.DMA((3,)),
            pltpu.SemaphoreType.DMA((4,)),
        ],
        compiler_params=pltpu.CompilerParams(
            collective_id=1, vmem_limit_bytes=60 * 1024 * 1024,
        ),
    )(x2, Wq, Wqr, k, v, kr, Wo)

    return y.reshape(1, S, D)


# device time: 111586 ns/iter; 1.3679x vs baseline; 1.2394x over previous
import jax
import jax.numpy as jnp
from jax import lax
from jax.experimental import pallas as pl
from jax.experimental.pallas import tpu as pltpu

S = 1024
D = 2048
DC_SH = 128
H = 16
DH = 128
DR = 32
SCALE = (DH + DR) ** -0.5
S4 = S // 4
HG = 4
NG = H // HG
GW = HG * DH

F32 = jnp.float32


def _dot(a, b):
    return jnp.dot(a, b, preferred_element_type=F32)


def _dot_t(a, b):
    return lax.dot_general(a, b, (((1,), (1,)), ((), ())),
                           preferred_element_type=F32)


def _body(x_ref, wdkv_ref, wuk_ref, wuv_ref, wuk_g_ref, wuv_g_ref,
          wkr_ref, wq_ref, wqr_ref, wo_ref, out_ref,
          c_mine_s, c_oth_s, wuk_o, wuv_o, kr_s, out_acc,
          w_send, w_recv, copy_sem, ag_send, ag_recv):
    my_x = lax.axis_index("x")
    my_y = lax.axis_index("y")
    p = my_x * 2 + my_y
    xnbr = (1 - my_x, my_y)
    peers = [(1 - my_x, my_y), (my_x, 1 - my_y), (1 - my_x, 1 - my_y)]
    peer_slots = [(1 - my_x) * 2 + my_y, my_x * 2 + (1 - my_y),
                  (1 - my_x) * 2 + (1 - my_y)]
    g = pl.program_id(0)
    barrier = pltpu.get_barrier_semaphore()

    def _wrdma(i, src, dst):
        return pltpu.make_async_remote_copy(
            src_ref=src, dst_ref=dst,
            send_sem=w_send.at[i], recv_sem=w_recv.at[i],
            device_id=xnbr, device_id_type=pl.DeviceIdType.MESH)

    def _w_descs():
        descs = [_wrdma(0, c_mine_s, c_oth_s)]
        for gg in range(NG):
            descs.append(_wrdma(1 + 2 * gg,
                                wuk_ref.at[:, gg * GW:(gg + 1) * GW],
                                wuk_o.at[gg]))
            descs.append(_wrdma(2 + 2 * gg,
                                wuv_ref.at[:, gg * GW:(gg + 1) * GW],
                                wuv_o.at[gg]))
        return descs

    @pl.when(g == 0)
    def _():
        for peer in peers:
            pl.semaphore_signal(barrier, inc=1, device_id=peer,
                                device_id_type=pl.DeviceIdType.MESH)
        pl.semaphore_wait(barrier, 3)
        c_mine_s[...] = _dot(x_ref[...], wdkv_ref[...])
        for d in _w_descs():
            d.start()
        kr_s[...] = _dot(x_ref[...], wkr_ref[...])

    x_mine = x_ref[pl.ds(p * S4, S4), :]
    q = _dot(x_mine, wq_ref[...])
    qr = _dot(x_mine, wqr_ref[...])

    @pl.when(g == 0)
    def _():
        _wrdma(0, c_mine_s, c_oth_s).wait_recv()

    for gg in range(NG):
        @pl.when(g == gg)
        def _(gg=gg):
            _wrdma(1 + 2 * gg, wuk_ref.at[:, gg * GW:(gg + 1) * GW],
                   wuk_o.at[gg]).wait_recv()
            _wrdma(2 + 2 * gg, wuv_ref.at[:, gg * GW:(gg + 1) * GW],
                   wuv_o.at[gg]).wait_recv()

    c_mine = c_mine_s[...]
    c_oth = c_oth_s[...]
    k_g = _dot(c_mine, wuk_g_ref[...]) + _dot(c_oth, wuk_o[g])
    v_g = _dot(c_mine, wuv_g_ref[...]) + _dot(c_oth, wuv_o[g])
    kr = kr_s[...]

    o_cols = []
    for i in range(HG):
        s = _dot_t(q[:, i * DH:(i + 1) * DH], k_g[:, i * DH:(i + 1) * DH])
        s = s + _dot_t(qr[:, i * DR:(i + 1) * DR], kr)
        s = s * SCALE
        m = jnp.max(s, axis=1, keepdims=True)
        pr = jnp.exp(s - m)
        pr = pr / jnp.sum(pr, axis=1, keepdims=True)
        o_cols.append(_dot(pr, v_g[:, i * DH:(i + 1) * DH]))
    proj = _dot(jnp.concatenate(o_cols, axis=1), wo_ref[...])

    @pl.when(g == 0)
    def _():
        out_acc[...] = proj

    @pl.when(g != 0)
    def _():
        out_acc[...] = out_acc[...] + proj

    @pl.when(g == NG - 1)
    def _():
        cp = pltpu.make_async_copy(out_acc, out_ref.at[p], copy_sem)
        cp.start()
        cp.wait()
        sends = []
        for i, peer in enumerate(peers):
            r = pltpu.make_async_remote_copy(
                src_ref=out_ref.at[p], dst_ref=out_ref.at[p],
                send_sem=ag_send.at[i], recv_sem=ag_recv.at[p],
                device_id=peer, device_id_type=pl.DeviceIdType.MESH)
            r.start()
            sends.append(r)
        for d in _w_descs():
            d.wait_send()
        for r in sends:
            r.wait_send()
        for i, qp in enumerate(peer_slots):
            r = pltpu.make_async_remote_copy(
                src_ref=out_ref.at[qp], dst_ref=out_ref.at[qp],
                send_sem=ag_send.at[i], recv_sem=ag_recv.at[qp],
                device_id=(my_x, my_y), device_id_type=pl.DeviceIdType.MESH)
            r.wait_recv()


def kernel(x, Wdkv, Wuk, Wuv, Wq, Wqr, Wkr, Wo):
    x2 = x.reshape(S, D)

    y = pl.pallas_call(
        _body,
        grid=(NG,),
        out_shape=jax.ShapeDtypeStruct((4, S4, D), F32),
        in_specs=[
            pl.BlockSpec((S, D), lambda g: (0, 0)),
            pl.BlockSpec((D, DC_SH), lambda g: (0, 0)),
            pl.BlockSpec((DC_SH, D), lambda g: (0, 0)),
            pl.BlockSpec((DC_SH, D), lambda g: (0, 0)),
            pl.BlockSpec((DC_SH, GW), lambda g: (0, g)),
            pl.BlockSpec((DC_SH, GW), lambda g: (0, g)),
            pl.BlockSpec((D, DR), lambda g: (0, 0)),
            pl.BlockSpec((D, GW), lambda g: (0, g)),
            pl.BlockSpec((D, HG * DR), lambda g: (0, g)),
            pl.BlockSpec((GW, D), lambda g: (g, 0)),
        ],
        out_specs=pl.BlockSpec(memory_space=pl.ANY),
        scratch_shapes=[
            pltpu.VMEM((S, DC_SH), F32),
            pltpu.VMEM((S, DC_SH), F32),
            pltpu.VMEM((NG, DC_SH, GW), F32),
            pltpu.VMEM((NG, DC_SH, GW), F32),
            pltpu.VMEM((S, DR), F32),
            pltpu.VMEM((S4, D), F32),
            pltpu.SemaphoreType.DMA((9,)),
            pltpu.SemaphoreType.DMA((9,)),
            pltpu.SemaphoreType.DMA,
            pltpu.SemaphoreType.DMA((3,)),
            pltpu.SemaphoreType.DMA((4,)),
        ],
        compiler_params=pltpu.CompilerParams(
            collective_id=0, vmem_limit_bytes=60 * 1024 * 1024,
        ),
    )(x2, Wdkv, Wuk, Wuv, Wuk, Wuv, Wkr, Wq, Wqr, Wo)

    return y.reshape(1, S, D)
